# Initial kernel scaffold; baseline (speedup 1.0000x reference)
import jax
import jax.numpy as jnp
from jax.experimental import pallas as pl


def kernel(h, W_proj, W_attn, edge_index):
    raise NotImplementedError("write your pallas kernel here")


# speedup vs baseline: 18.2374x; 18.2374x over previous
import functools

import jax
import jax.numpy as jnp
from jax import lax
from jax.experimental import pallas as pl
from jax.experimental.pallas import tpu as pltpu
from jax.experimental.pallas import tpu_sc as plsc

N_NODES = 10000
N_EDGES = 320000
DIM = 128

NC = 2
NS = 16
NW = NC * NS
EPW = N_EDGES // NW
CHUNK = 80
NCHUNK = EPW // CHUNK
N_ACC = 10240
RPW = N_ACC // NS


def _proj_body(h_ref, wp_ref, wa_ref, wh_ref, sd_ref):
    wh = lax.dot_general(h_ref[...], wp_ref[...],
                         (((1,), (1,)), ((), ())),
                         preferred_element_type=jnp.float32)
    wh_ref[...] = wh
    sd_ref[...] = lax.dot_general(wh, wa_ref[...],
                                  (((1,), (1,)), ((), ())),
                                  preferred_element_type=jnp.float32)


def _proj(h, W_proj, wa2):
    blk = 1000
    grid = N_NODES // blk
    return pl.pallas_call(
        _proj_body,
        grid=(grid,),
        in_specs=[
            pl.BlockSpec((blk, DIM), lambda i: (i, 0)),
            pl.BlockSpec((DIM, DIM), lambda i: (0, 0)),
            pl.BlockSpec((2, DIM), lambda i: (0, 0)),
        ],
        out_specs=[
            pl.BlockSpec((blk, DIM), lambda i: (i, 0)),
            pl.BlockSpec((blk, 2), lambda i: (i, 0)),
        ],
        out_shape=[
            jax.ShapeDtypeStruct((N_NODES, DIM), jnp.float32),
            jax.ShapeDtypeStruct((N_NODES, 2), jnp.float32),
        ],
    )(h, W_proj, wa2)


def _edge_body(wh_hbm, s_hbm, d_hbm, src_hbm, dst_hbm,
               acc_hbm, den_hbm,
               s_tab, d_tab, sbuf, dbuf, wbuf, rows, den_tab, acc, sem):
    sub = lax.axis_index("s")
    core = lax.axis_index("c")
    wid = sub * NC + core

    def _zrow(i, _):
        for j in range(DIM // 16):
            rows[i, pl.ds(j * 16, 16)] = jnp.zeros((16,), jnp.float32)
        return 0
    lax.fori_loop(0, CHUNK, _zrow, 0)
    for t in range(RPW // CHUNK):
        pltpu.sync_copy(rows, acc.at[pl.ds(sub * RPW + t * CHUNK, CHUNK)])

    def _zden(i, _):
        den_tab[0, pl.ds(i * 16, 16)] = jnp.zeros((16,), jnp.float32)
        return 0
    lax.fori_loop(0, N_ACC // 16, _zden, 0)
    pltpu.sync_copy(s_hbm, s_tab)
    pltpu.sync_copy(d_hbm, d_tab)

    plsc.subcore_barrier()

    ebase = wid * EPW
    zero16 = jnp.zeros((16,), jnp.int32)

    def _chunk(c, _):
        base = ebase + c * CHUNK
        pltpu.sync_copy(src_hbm.at[pl.ds(base, CHUNK)], sbuf)
        pltpu.sync_copy(dst_hbm.at[pl.ds(base, CHUNK)], dbuf)
        pltpu.async_copy(wh_hbm.at[sbuf], rows, sem).wait()
        for g in range(CHUNK // 16):
            isrc = sbuf[pl.ds(g * 16, 16)]
            idst = dbuf[pl.ds(g * 16, 16)]
            sv = plsc.load_gather(s_tab, [isrc])
            dv = plsc.load_gather(d_tab, [idst])
            w = jnp.exp(jnp.maximum(sv + dv, 0.0))
            wbuf[pl.ds(g * 16, 16)] = w
            plsc.addupdate_scatter(den_tab, [zero16, idst], w)

        def _scale(g, _):
            wv = wbuf[pl.ds(g * 16, 16)]
            for l in range(16):
                ws = wv[l]
                e = g * 16 + l
                for j in range(DIM // 16):
                    rows[e, pl.ds(j * 16, 16)] = rows[e, pl.ds(j * 16, 16)] * ws
            return 0
        lax.fori_loop(0, CHUNK // 16, _scale, 0)

        pltpu.sync_copy(rows, acc.at[dbuf], add=True)
        return 0

    lax.fori_loop(0, NCHUNK, _chunk, 0)

    plsc.subcore_barrier()

    pltpu.sync_copy(acc.at[pl.ds(sub * RPW, RPW)],
                    acc_hbm.at[core, pl.ds(sub * RPW, RPW)])
    pltpu.sync_copy(den_tab, den_hbm.at[wid])


def _edge(Wh, s, d, src, dst):
    mesh = plsc.VectorSubcoreMesh(core_axis_name="c", subcore_axis_name="s")
    fn = functools.partial(
        pl.kernel,
        mesh=mesh,
        compiler_params=pltpu.CompilerParams(needs_layout_passes=False),
        out_type=[
            jax.ShapeDtypeStruct((NC, N_ACC, DIM), jnp.float32),
            jax.ShapeDtypeStruct((NW, 1, N_ACC), jnp.float32),
        ],
        scratch_types=[
            pltpu.VMEM((N_NODES,), jnp.float32),
            pltpu.VMEM((N_NODES,), jnp.float32),
            pltpu.VMEM((CHUNK,), jnp.int32),
            pltpu.VMEM((CHUNK,), jnp.int32),
            pltpu.VMEM((CHUNK,), jnp.float32),
            pltpu.VMEM((CHUNK, DIM), jnp.float32),
            pltpu.VMEM((1, N_ACC), jnp.float32),
            pltpu.VMEM_SHARED((N_ACC, DIM), jnp.float32),
            pltpu.SemaphoreType.DMA,
        ],
    )(_edge_body)
    return fn(Wh, s, d, src, dst)


def _finish_body(acc_ref, den_ref, out_ref):
    num = acc_ref[0] + acc_ref[1]
    den = jnp.maximum(jnp.sum(den_ref[...], axis=0), 1e-16)
    out_ref[...] = num / den[:, None]


def _finish(acc, den):
    blk = 1280
    grid = N_ACC // blk
    return pl.pallas_call(
        _finish_body,
        grid=(grid,),
        in_specs=[
            pl.BlockSpec((NC, blk, DIM), lambda i: (0, i, 0)),
            pl.BlockSpec((NW, blk), lambda i: (0, i)),
        ],
        out_specs=pl.BlockSpec((blk, DIM), lambda i: (i, 0)),
        out_shape=jax.ShapeDtypeStruct((N_NODES, DIM), jnp.float32),
    )(acc, den)


def kernel(h, W_proj, W_attn, edge_index):
    ei = edge_index.astype(jnp.int32)
    src = ei[0]
    dst = ei[1]
    wa2 = W_attn.reshape(2, DIM)
    Wh, sd = _proj(h, W_proj, wa2)
    s = sd[:, 0]
    d = sd[:, 1]
    acc, den = _edge(Wh, s, d, src, dst)
    return _finish(acc, den.reshape(NW, N_ACC))
